# Initial kernel scaffold; baseline (speedup 1.0000x reference)
import jax
import jax.numpy as jnp
from jax.experimental import pallas as pl


def kernel(msg, curr_emb):
    raise NotImplementedError("write your pallas kernel here")


# speedup vs baseline: 3.1805x; 3.1805x over previous
import functools

import jax
import jax.numpy as jnp
from jax import lax
from jax.experimental import pallas as pl
from jax.experimental.pallas import tpu as pltpu
from jax.experimental.pallas import tpu_sc as plsc

_NUM_WORKERS = 32
_R = 40


def _sc_concat(msg, curr_emb):
    n, e, d = msg.shape
    num_chunks = n // _R
    assert num_chunks * _R == n
    full_iters = num_chunks // _NUM_WORKERS
    rem = num_chunks - full_iters * _NUM_WORKERS
    mesh = plsc.VectorSubcoreMesh(core_axis_name="c", subcore_axis_name="s")
    buf = pltpu.VMEM((_R, 1, d), jnp.float32)

    @functools.partial(
        pl.kernel,
        out_type=jax.ShapeDtypeStruct((n, 4 * d), jnp.float32),
        mesh=mesh,
        scratch_types=[
            [[buf, buf, buf, buf], [buf, buf, buf, buf]],
            pltpu.SemaphoreType.DMA,
            pltpu.SemaphoreType.DMA,
            pltpu.SemaphoreType.DMA,
            pltpu.SemaphoreType.DMA,
        ],
    )
    def k(msg_hbm, cur_hbm, out_hbm, bufs, sg0, sg1, ss0, ss1):
        wid = lax.axis_index("s") * 2 + lax.axis_index("c")
        sgs = (sg0, sg1)
        sss = (ss0, ss1)

        def chunk_rows(j):
            return pl.ds((wid + j * _NUM_WORKERS) * _R, _R)

        def fire_gather(j, s):
            rows = chunk_rows(j)
            descs = [
                pltpu.async_copy(cur_hbm.at[rows, pl.ds(0, 1)], bufs[s][0], sgs[s])
            ]
            for i in range(3):
                descs.append(
                    pltpu.async_copy(
                        msg_hbm.at[rows, pl.ds(i, 1)], bufs[s][i + 1], sgs[s]
                    )
                )
            return descs

        def fire_scatter(j, s):
            rows = chunk_rows(j)
            return [
                pltpu.async_copy(
                    bufs[s][i].at[:, 0],
                    out_hbm.at[rows, pl.ds(i * d, d)],
                    sss[s],
                )
                for i in range(4)
            ]

        pend_g = {0: [], 1: []}
        pend_s = {0: [], 1: []}
        pend_g[0] = fire_gather(0, 0)
        for j in range(full_iters):
            s = j % 2
            nxt = (j + 1) % 2
            if j + 1 < full_iters:
                for de in pend_s[nxt]:
                    de.wait()
                pend_s[nxt] = []
                pend_g[nxt] = fire_gather(j + 1, nxt)
            for de in pend_g[s]:
                de.wait()
            pend_g[s] = []
            pend_s[s] = fire_scatter(j, s)
        for s in (0, 1):
            for de in pend_s[s]:
                de.wait()
            pend_s[s] = []
        if rem:
            @pl.when(wid < rem)
            def _():
                gd = fire_gather(full_iters, 0)
                for de in gd:
                    de.wait()
                sd = fire_scatter(full_iters, 0)
                for de in sd:
                    de.wait()

    return k(msg, curr_emb)


def kernel(msg, curr_emb):
    return _sc_concat(msg, curr_emb)


# speedup vs baseline: 3.2242x; 1.0137x over previous
import functools

import jax
import jax.numpy as jnp
from jax import lax
from jax.experimental import pallas as pl
from jax.experimental.pallas import tpu as pltpu
from jax.experimental.pallas import tpu_sc as plsc

_NUM_WORKERS = 32
_R = 40
_NSET = 3


def _sc_concat(msg, curr_emb):
    n, e, d = msg.shape
    num_chunks = n // _R
    assert num_chunks * _R == n
    full_iters = num_chunks // _NUM_WORKERS
    rem = num_chunks - full_iters * _NUM_WORKERS
    mesh = plsc.VectorSubcoreMesh(core_axis_name="c", subcore_axis_name="s")
    buf = pltpu.VMEM((_R, 1, d), jnp.float32)
    sem = pltpu.SemaphoreType.DMA

    @functools.partial(
        pl.kernel,
        out_type=jax.ShapeDtypeStruct((n, 4 * d), jnp.float32),
        mesh=mesh,
        scratch_types=[
            [[buf] * 4 for _ in range(_NSET)],
            [sem] * _NSET,
            [sem] * _NSET,
        ],
    )
    def k(msg_hbm, cur_hbm, out_hbm, bufs, sgs, sss):
        wid = lax.axis_index("s") * 2 + lax.axis_index("c")

        def chunk_rows(j):
            return pl.ds((wid + j * _NUM_WORKERS) * _R, _R)

        def fire_gather(j, s):
            rows = chunk_rows(j)
            descs = [
                pltpu.async_copy(cur_hbm.at[rows, pl.ds(0, 1)], bufs[s][0], sgs[s])
            ]
            for i in range(3):
                descs.append(
                    pltpu.async_copy(
                        msg_hbm.at[rows, pl.ds(i, 1)], bufs[s][i + 1], sgs[s]
                    )
                )
            return descs

        def fire_scatter(j, s):
            rows = chunk_rows(j)
            return [
                pltpu.async_copy(
                    bufs[s][i].at[:, 0],
                    out_hbm.at[rows, pl.ds(i * d, d)],
                    sss[s],
                )
                for i in range(4)
            ]

        pend_g = {s: [] for s in range(_NSET)}
        pend_s = {s: [] for s in range(_NSET)}
        for j in range(min(_NSET - 1, full_iters)):
            pend_g[j % _NSET] = fire_gather(j, j % _NSET)
        for j in range(full_iters):
            s = j % _NSET
            pre = j + _NSET - 1
            if pre < full_iters:
                sp = pre % _NSET
                for de in pend_s[sp]:
                    de.wait()
                pend_s[sp] = []
                pend_g[sp] = fire_gather(pre, sp)
            for de in pend_g[s]:
                de.wait()
            pend_g[s] = []
            pend_s[s] = fire_scatter(j, s)
        for s in range(_NSET):
            for de in pend_s[s]:
                de.wait()
            pend_s[s] = []
        if rem:
            @pl.when(wid < rem)
            def _():
                gd = fire_gather(full_iters, 0)
                for de in gd:
                    de.wait()
                sd = fire_scatter(full_iters, 0)
                for de in sd:
                    de.wait()

    return k(msg, curr_emb)


def kernel(msg, curr_emb):
    return _sc_concat(msg, curr_emb)


# speedup vs baseline: 3.2244x; 1.0001x over previous
import functools

import jax
import jax.numpy as jnp
from jax import lax
from jax.experimental import pallas as pl
from jax.experimental.pallas import tpu as pltpu
from jax.experimental.pallas import tpu_sc as plsc

_NUM_WORKERS = 32
_R = 40
_NSET = 3


def _sc_concat(msg, curr_emb):
    n, e, d = msg.shape
    num_chunks = n // _R
    assert num_chunks * _R == n
    full_iters = num_chunks // _NUM_WORKERS
    rem = num_chunks - full_iters * _NUM_WORKERS
    mesh = plsc.VectorSubcoreMesh(core_axis_name="c", subcore_axis_name="s")
    buf = pltpu.VMEM((_R, 1, d), jnp.float32)
    sem = pltpu.SemaphoreType.DMA

    @functools.partial(
        pl.kernel,
        out_type=jax.ShapeDtypeStruct((n, 4 * d), jnp.float32),
        mesh=mesh,
        compiler_params=pltpu.CompilerParams(skip_device_barrier=True),
        scratch_types=[
            [[buf] * 4 for _ in range(_NSET)],
            [sem] * _NSET,
            [sem] * _NSET,
        ],
    )
    def k(msg_hbm, cur_hbm, out_hbm, bufs, sgs, sss):
        wid = lax.axis_index("s") * 2 + lax.axis_index("c")

        def chunk_rows(j):
            return pl.ds((wid + j * _NUM_WORKERS) * _R, _R)

        def fire_gather(j, s):
            rows = chunk_rows(j)
            descs = [
                pltpu.async_copy(cur_hbm.at[rows, pl.ds(0, 1)], bufs[s][0], sgs[s])
            ]
            for i in range(3):
                descs.append(
                    pltpu.async_copy(
                        msg_hbm.at[rows, pl.ds(i, 1)], bufs[s][i + 1], sgs[s]
                    )
                )
            return descs

        def fire_scatter(j, s):
            rows = chunk_rows(j)
            return [
                pltpu.async_copy(
                    bufs[s][i].at[:, 0],
                    out_hbm.at[rows, pl.ds(i * d, d)],
                    sss[s],
                )
                for i in range(4)
            ]

        pend_g = {s: [] for s in range(_NSET)}
        pend_s = {s: [] for s in range(_NSET)}
        for j in range(min(_NSET - 1, full_iters)):
            pend_g[j % _NSET] = fire_gather(j, j % _NSET)
        for j in range(full_iters):
            s = j % _NSET
            pre = j + _NSET - 1
            if pre < full_iters:
                sp = pre % _NSET
                for de in pend_s[sp]:
                    de.wait()
                pend_s[sp] = []
                pend_g[sp] = fire_gather(pre, sp)
            for de in pend_g[s]:
                de.wait()
            pend_g[s] = []
            pend_s[s] = fire_scatter(j, s)
        for s in range(_NSET):
            for de in pend_s[s]:
                de.wait()
            pend_s[s] = []
        if rem:
            @pl.when(wid < rem)
            def _():
                gd = fire_gather(full_iters, 0)
                for de in gd:
                    de.wait()
                sd = fire_scatter(full_iters, 0)
                for de in sd:
                    de.wait()

    return k(msg, curr_emb)


def kernel(msg, curr_emb):
    return _sc_concat(msg, curr_emb)


# speedup vs baseline: 3.2280x; 1.0011x over previous
import functools

import jax
import jax.numpy as jnp
from jax import lax
from jax.experimental import pallas as pl
from jax.experimental.pallas import tpu as pltpu
from jax.experimental.pallas import tpu_sc as plsc

_NUM_WORKERS = 32
_R = 40
_NSET = 3


def _sc_concat(msg, curr_emb):
    n, e, d = msg.shape
    num_chunks = n // _R
    assert num_chunks * _R == n
    full_iters = num_chunks // _NUM_WORKERS
    rem = num_chunks - full_iters * _NUM_WORKERS
    mesh = plsc.VectorSubcoreMesh(core_axis_name="c", subcore_axis_name="s")
    buf = pltpu.VMEM((_R, 1, d), jnp.float32)
    sem = pltpu.SemaphoreType.DMA

    @functools.partial(
        pl.kernel,
        out_type=jax.ShapeDtypeStruct((n, 4 * d), jnp.float32),
        mesh=mesh,
        scratch_types=[
            [[buf] * 4 for _ in range(_NSET)],
            [sem] * _NSET,
            [sem] * _NSET,
        ],
    )
    def k(msg_hbm, cur_hbm, out_hbm, bufs, sgs, sss):
        wid = lax.axis_index("s") * 2 + lax.axis_index("c")

        def chunk_rows(j):
            return pl.ds((wid + j * _NUM_WORKERS) * _R, _R)

        def fire_gather(j, s):
            rows = chunk_rows(j)
            descs = [
                pltpu.async_copy(cur_hbm.at[rows, pl.ds(0, 1)], bufs[s][0], sgs[s])
            ]
            for i in range(3):
                descs.append(
                    pltpu.async_copy(
                        msg_hbm.at[rows, pl.ds(i, 1)], bufs[s][i + 1], sgs[s]
                    )
                )
            return descs

        def fire_scatter(j, s):
            rows = chunk_rows(j)
            return [
                pltpu.async_copy(
                    bufs[s][i].at[:, 0],
                    out_hbm.at[rows, pl.ds(i * d, d)],
                    sss[s],
                )
                for i in range(4)
            ]

        pend_g = {s: [] for s in range(_NSET)}
        pend_s = {s: [] for s in range(_NSET)}
        for j in range(min(_NSET - 1, full_iters)):
            pend_g[j % _NSET] = fire_gather(j, j % _NSET)
        for j in range(full_iters):
            s = j % _NSET
            pre = j + _NSET - 1
            if pre < full_iters:
                sp = pre % _NSET
                for de in pend_s[sp]:
                    de.wait()
                pend_s[sp] = []
                pend_g[sp] = fire_gather(pre, sp)
            for de in pend_g[s]:
                de.wait()
            pend_g[s] = []
            pend_s[s] = fire_scatter(j, s)
        for s in range(_NSET):
            for de in pend_s[s]:
                de.wait()
            pend_s[s] = []
        if rem:
            @pl.when(wid < rem)
            def _():
                gd = fire_gather(full_iters, 0)
                for de in gd:
                    de.wait()
                sd = fire_scatter(full_iters, 0)
                for de in sd:
                    de.wait()

    return k(msg, curr_emb)


def kernel(msg, curr_emb):
    return _sc_concat(msg, curr_emb)


# speedup vs baseline: 3.3540x; 1.0390x over previous
import functools

import jax
import jax.numpy as jnp
from jax import lax
from jax.experimental import pallas as pl
from jax.experimental.pallas import tpu as pltpu
from jax.experimental.pallas import tpu_sc as plsc

_NUM_WORKERS = 32
_R = 40
_NSET = 3


def _sc_concat(msg, curr_emb):
    n, e, d = msg.shape
    num_chunks = n // _R
    assert num_chunks * _R == n
    full_iters = num_chunks // _NUM_WORKERS
    rem = num_chunks - full_iters * _NUM_WORKERS
    mesh = plsc.VectorSubcoreMesh(core_axis_name="c", subcore_axis_name="s")
    cbuf = pltpu.VMEM((_R, 1, d), jnp.float32)
    mbuf = pltpu.VMEM((_R, 3 * d), jnp.float32)
    sem = pltpu.SemaphoreType.DMA

    @functools.partial(
        pl.kernel,
        out_type=jax.ShapeDtypeStruct((n, 4 * d), jnp.float32),
        mesh=mesh,
        scratch_types=[
            [[cbuf, mbuf] for _ in range(_NSET)],
            [sem] * _NSET,
            [sem] * _NSET,
        ],
    )
    def k(msg_hbm, cur_hbm, out_hbm, bufs, sgs, sss):
        wid = lax.axis_index("s") * 2 + lax.axis_index("c")

        def chunk_rows(j):
            return pl.ds((wid + j * _NUM_WORKERS) * _R, _R)

        def fire_gather(j, s):
            rows = chunk_rows(j)
            descs = [
                pltpu.async_copy(cur_hbm.at[rows, pl.ds(0, 1)], bufs[s][0], sgs[s])
            ]
            for i in range(3):
                descs.append(
                    pltpu.async_copy(
                        msg_hbm.at[rows, i],
                        bufs[s][1].at[:, pl.ds(i * d, d)],
                        sgs[s],
                    )
                )
            return descs

        def fire_scatter(j, s):
            rows = chunk_rows(j)
            return [
                pltpu.async_copy(
                    bufs[s][0].at[:, 0],
                    out_hbm.at[rows, pl.ds(0, d)],
                    sss[s],
                ),
                pltpu.async_copy(
                    bufs[s][1],
                    out_hbm.at[rows, pl.ds(d, 3 * d)],
                    sss[s],
                ),
            ]

        pend_g = {s: [] for s in range(_NSET)}
        pend_s = {s: [] for s in range(_NSET)}
        for j in range(min(_NSET - 1, full_iters)):
            pend_g[j % _NSET] = fire_gather(j, j % _NSET)
        for j in range(full_iters):
            s = j % _NSET
            pre = j + _NSET - 1
            if pre < full_iters:
                sp = pre % _NSET
                for de in pend_s[sp]:
                    de.wait()
                pend_s[sp] = []
                pend_g[sp] = fire_gather(pre, sp)
            for de in pend_g[s]:
                de.wait()
            pend_g[s] = []
            pend_s[s] = fire_scatter(j, s)
        for s in range(_NSET):
            for de in pend_s[s]:
                de.wait()
            pend_s[s] = []
        if rem:
            @pl.when(wid < rem)
            def _():
                gd = fire_gather(full_iters, 0)
                for de in gd:
                    de.wait()
                sd = fire_scatter(full_iters, 0)
                for de in sd:
                    de.wait()

    return k(msg, curr_emb)


def kernel(msg, curr_emb):
    return _sc_concat(msg, curr_emb)


# speedup vs baseline: 3.3678x; 1.0041x over previous
import functools

import jax
import jax.numpy as jnp
from jax import lax
from jax.experimental import pallas as pl
from jax.experimental.pallas import tpu as pltpu
from jax.experimental.pallas import tpu_sc as plsc

_NUM_WORKERS = 32
_R = 40
_NSET = 3


def _sc_concat(msg, curr_emb):
    n, e, d = msg.shape
    num_chunks = n // _R
    assert num_chunks * _R == n
    full_iters = num_chunks // _NUM_WORKERS
    rem = num_chunks - full_iters * _NUM_WORKERS
    mesh = plsc.VectorSubcoreMesh(core_axis_name="c", subcore_axis_name="s")
    rbuf = pltpu.VMEM((_R, 4 * d), jnp.float32)
    sem = pltpu.SemaphoreType.DMA

    @functools.partial(
        pl.kernel,
        out_type=jax.ShapeDtypeStruct((n, 4 * d), jnp.float32),
        mesh=mesh,
        scratch_types=[
            [rbuf] * _NSET,
            [sem] * _NSET,
            [sem] * _NSET,
        ],
    )
    def k(msg_hbm, cur_hbm, out_hbm, bufs, sgs, sss):
        wid = lax.axis_index("s") * 2 + lax.axis_index("c")

        def chunk_rows(j):
            return pl.ds((wid + j * _NUM_WORKERS) * _R, _R)

        def fire_gather(j, s):
            rows = chunk_rows(j)
            descs = [
                pltpu.async_copy(
                    cur_hbm.at[rows, 0], bufs[s].at[:, pl.ds(0, d)], sgs[s]
                )
            ]
            for i in range(3):
                descs.append(
                    pltpu.async_copy(
                        msg_hbm.at[rows, i],
                        bufs[s].at[:, pl.ds((i + 1) * d, d)],
                        sgs[s],
                    )
                )
            return descs

        def fire_scatter(j, s):
            rows = chunk_rows(j)
            return [
                pltpu.async_copy(bufs[s], out_hbm.at[rows], sss[s]),
            ]

        pend_g = {s: [] for s in range(_NSET)}
        pend_s = {s: [] for s in range(_NSET)}
        for j in range(min(_NSET - 1, full_iters)):
            pend_g[j % _NSET] = fire_gather(j, j % _NSET)
        for j in range(full_iters):
            s = j % _NSET
            pre = j + _NSET - 1
            if pre < full_iters:
                sp = pre % _NSET
                for de in pend_s[sp]:
                    de.wait()
                pend_s[sp] = []
                pend_g[sp] = fire_gather(pre, sp)
            for de in pend_g[s]:
                de.wait()
            pend_g[s] = []
            pend_s[s] = fire_scatter(j, s)
        for s in range(_NSET):
            for de in pend_s[s]:
                de.wait()
            pend_s[s] = []
        if rem:
            @pl.when(wid < rem)
            def _():
                gd = fire_gather(full_iters, 0)
                for de in gd:
                    de.wait()
                sd = fire_scatter(full_iters, 0)
                for de in sd:
                    de.wait()

    return k(msg, curr_emb)


def kernel(msg, curr_emb):
    return _sc_concat(msg, curr_emb)
